# baseline (speedup 1.0000x reference)
import functools

import jax
import jax.numpy as jnp
from jax import lax
from jax.experimental import pallas as pl
from jax.experimental.pallas import tpu as pltpu
from jax.experimental.pallas import tpu_sc as plsc

N_TOKENS = 16384
IN_FEATURES = 4096
N_EXPERTS = 64
TOP_K = 8
BM = 1024

NW = 32
TPW = N_TOKENS // NW
NG = TPW // 16


def _scores_kernel(x_ref, w_ref, s_out_ref):
    lt = jax.lax.dot_general(
        w_ref[...],
        x_ref[...],
        (((1,), (1,)), ((), ())),
        preferred_element_type=jnp.float32,
    )
    m = jnp.max(lt, axis=0, keepdims=True)
    e = jnp.exp(lt - m)
    z = jnp.sum(e, axis=0, keepdims=True)
    s_out_ref[...] = e / z


def _tc_scores(x, W):
    grid = (N_TOKENS // BM,)
    return pl.pallas_call(
        _scores_kernel,
        grid=grid,
        in_specs=[
            pl.BlockSpec((BM, IN_FEATURES), lambda i: (i, 0)),
            pl.BlockSpec((N_EXPERTS, IN_FEATURES), lambda i: (0, 0)),
        ],
        out_specs=pl.BlockSpec((N_EXPERTS, BM), lambda i: (0, i)),
        out_shape=jax.ShapeDtypeStruct((N_EXPERTS, N_TOKENS), jnp.float32),
    )(x, W)


def _sc_topk_body(s_hbm, w_out_hbm, i_out_hbm, slab, outw, outi):
    wid = lax.axis_index("s") * 2 + lax.axis_index("c")
    t0 = wid * TPW
    pltpu.sync_copy(s_hbm.at[:, pl.ds(t0, TPW)], slab)

    def group(g, _):
        base = g * 16
        m_prev = None
        idx_prev = None
        for j in range(TOP_K):
            m = jnp.full((16,), -jnp.inf, jnp.float32)
            for e in range(N_EXPERTS):
                v = slab[e, pl.ds(base, 16)]
                if j > 0:
                    elig = (v < m_prev) | ((v == m_prev) & (e > idx_prev))
                    v = jnp.where(elig, v, -jnp.inf)
                m = jnp.maximum(m, v)
            idx = jnp.full((16,), N_EXPERTS, jnp.int32)
            for e in reversed(range(N_EXPERTS)):
                v = slab[e, pl.ds(base, 16)]
                c = v == m
                if j > 0:
                    c = c & ((v < m_prev) | ((v == m_prev) & (e > idx_prev)))
                idx = jnp.where(c, jnp.int32(e), idx)
            outw[j, pl.ds(base, 16)] = m
            outi[j, pl.ds(base, 16)] = idx
            m_prev, idx_prev = m, idx
        return ()

    lax.fori_loop(0, NG, group, ())

    pltpu.sync_copy(outw, w_out_hbm.at[:, pl.ds(t0, TPW)])
    pltpu.sync_copy(outi, i_out_hbm.at[:, pl.ds(t0, TPW)])


def _sc_topk(scores_t):
    mesh = plsc.VectorSubcoreMesh(core_axis_name="c", subcore_axis_name="s")

    @functools.partial(
        pl.kernel,
        mesh=mesh,
        out_type=[
            jax.ShapeDtypeStruct((TOP_K, N_TOKENS), jnp.float32),
            jax.ShapeDtypeStruct((TOP_K, N_TOKENS), jnp.int32),
        ],
        scratch_types=[
            pltpu.VMEM((N_EXPERTS, TPW), jnp.float32),
            pltpu.VMEM((TOP_K, TPW), jnp.float32),
            pltpu.VMEM((TOP_K, TPW), jnp.int32),
        ],
    )
    def k(s_hbm, w_out_hbm, i_out_hbm, slab, outw, outi):
        _sc_topk_body(s_hbm, w_out_hbm, i_out_hbm, slab, outw, outi)

    return k(scores_t)


def kernel(x, W):
    scores_t = _tc_scores(x, W)
    w_t, i_t = _sc_topk(scores_t)
    return w_t.T, i_t.T


# speedup vs baseline: 2.3205x; 2.3205x over previous
import jax
import jax.numpy as jnp
from jax.experimental import pallas as pl

N_TOKENS = 16384
IN_FEATURES = 4096
N_EXPERTS = 64
TOP_K = 8
BM = 1024


def _gate_kernel(x_ref, w_ref, w_out_ref, i_out_ref):
    lt = jax.lax.dot_general(
        w_ref[...],
        x_ref[...],
        (((1,), (1,)), ((), ())),
        preferred_element_type=jnp.float32,
    )
    iota = jax.lax.broadcasted_iota(jnp.int32, (N_EXPERTS, BM), 0)

    l = lt
    tops = []
    idxs = []
    for j in range(TOP_K):
        cur = jnp.max(l, axis=0, keepdims=True)
        hit = l == cur
        idx = jnp.min(jnp.where(hit, iota, N_EXPERTS), axis=0, keepdims=True)
        tops.append(cur)
        idxs.append(idx)
        l = jnp.where(hit, float("-inf"), l)

    m = tops[0]
    z = jnp.sum(jnp.exp(lt - m), axis=0, keepdims=True)
    for j in range(TOP_K):
        w_out_ref[j : j + 1, :] = jnp.exp(tops[j] - m) / z
        i_out_ref[j : j + 1, :] = idxs[j]


def kernel(x, W):
    grid = (N_TOKENS // BM,)
    weights_t, indices_t = pl.pallas_call(
        _gate_kernel,
        grid=grid,
        in_specs=[
            pl.BlockSpec((BM, IN_FEATURES), lambda i: (i, 0)),
            pl.BlockSpec((N_EXPERTS, IN_FEATURES), lambda i: (0, 0)),
        ],
        out_specs=[
            pl.BlockSpec((TOP_K, BM), lambda i: (0, i)),
            pl.BlockSpec((TOP_K, BM), lambda i: (0, i)),
        ],
        out_shape=[
            jax.ShapeDtypeStruct((TOP_K, N_TOKENS), jnp.float32),
            jax.ShapeDtypeStruct((TOP_K, N_TOKENS), jnp.int32),
        ],
    )(x, W)
    return weights_t.T, indices_t.T


# speedup vs baseline: 2.3303x; 1.0042x over previous
import jax
import jax.numpy as jnp
from jax.experimental import pallas as pl

N_TOKENS = 16384
IN_FEATURES = 4096
N_EXPERTS = 64
TOP_K = 8
BM = 1024


def _gate_kernel(x_ref, w_ref, w_out_ref, i_out_ref):
    lt = jax.lax.dot_general(
        w_ref[...],
        x_ref[...],
        (((1,), (1,)), ((), ())),
        preferred_element_type=jnp.float32,
    )
    m = jnp.max(lt, axis=0, keepdims=True)
    e = jnp.exp(lt - m)
    z = jnp.sum(e, axis=0, keepdims=True)
    s = e / z

    iota = jax.lax.broadcasted_iota(jnp.int32, (N_EXPERTS, BM), 0)
    for j in range(TOP_K):
        cur = jnp.max(s, axis=0, keepdims=True)
        hit = s == cur
        idx = jnp.min(jnp.where(hit, iota, N_EXPERTS), axis=0, keepdims=True)
        w_out_ref[j : j + 1, :] = cur
        i_out_ref[j : j + 1, :] = idx
        s = jnp.where(iota == idx, -1.0, s)


def kernel(x, W):
    grid = (N_TOKENS // BM,)
    weights_t, indices_t = pl.pallas_call(
        _gate_kernel,
        grid=grid,
        in_specs=[
            pl.BlockSpec((BM, IN_FEATURES), lambda i: (i, 0)),
            pl.BlockSpec((N_EXPERTS, IN_FEATURES), lambda i: (0, 0)),
        ],
        out_specs=[
            pl.BlockSpec((TOP_K, BM), lambda i: (0, i)),
            pl.BlockSpec((TOP_K, BM), lambda i: (0, i)),
        ],
        out_shape=[
            jax.ShapeDtypeStruct((TOP_K, N_TOKENS), jnp.float32),
            jax.ShapeDtypeStruct((TOP_K, N_TOKENS), jnp.int32),
        ],
    )(x, W)
    return weights_t.T, indices_t.T


# speedup vs baseline: 2.3325x; 1.0009x over previous
import jax
import jax.numpy as jnp
from jax.experimental import pallas as pl
from jax.experimental.pallas import tpu as pltpu

N_TOKENS = 16384
IN_FEATURES = 4096
N_EXPERTS = 64
TOP_K = 8
BM = 1024


def _gate_kernel(x_ref, w_ref, w_out_ref, i_out_ref):
    lt = jax.lax.dot_general(
        w_ref[...],
        x_ref[...],
        (((1,), (1,)), ((), ())),
        preferred_element_type=jnp.float32,
    )
    m = jnp.max(lt, axis=0, keepdims=True)
    e = jnp.exp(lt - m)
    z = jnp.sum(e, axis=0, keepdims=True)
    s = e / z

    iota = jax.lax.broadcasted_iota(jnp.int32, (N_EXPERTS, BM), 0)
    for j in range(TOP_K):
        cur = jnp.max(s, axis=0, keepdims=True)
        hit = s == cur
        idx = jnp.min(jnp.where(hit, iota, N_EXPERTS), axis=0, keepdims=True)
        w_out_ref[j : j + 1, :] = cur
        i_out_ref[j : j + 1, :] = idx
        s = jnp.where(iota == idx, -1.0, s)


def kernel(x, W):
    grid = (N_TOKENS // BM,)
    weights_t, indices_t = pl.pallas_call(
        _gate_kernel,
        grid=grid,
        in_specs=[
            pl.BlockSpec((BM, IN_FEATURES), lambda i: (i, 0)),
            pl.BlockSpec((N_EXPERTS, IN_FEATURES), lambda i: (0, 0)),
        ],
        out_specs=[
            pl.BlockSpec((TOP_K, BM), lambda i: (0, i)),
            pl.BlockSpec((TOP_K, BM), lambda i: (0, i)),
        ],
        out_shape=[
            jax.ShapeDtypeStruct((TOP_K, N_TOKENS), jnp.float32),
            jax.ShapeDtypeStruct((TOP_K, N_TOKENS), jnp.int32),
        ],
        compiler_params=pltpu.CompilerParams(
            dimension_semantics=("parallel",)
        ),
    )(x, W)
    return weights_t.T, indices_t.T
